# baseline (speedup 1.0000x reference)
import functools

import jax
import jax.numpy as jnp
from jax import lax
from jax.experimental import pallas as pl
from jax.experimental.pallas import tpu as pltpu
from jax.experimental.pallas import tpu_sc as plsc

IN_F = 768
NE = 8
BETA_F = 1.0e6
V_IMP_F = 0.1

NS = 16
BT = 256

SC_CORES = 2
SC_SUBCORES = 16
SC_LANES = 16
NW = SC_CORES * SC_SUBCORES


def _tc_body(*refs, nb):
    x_refs = refs[:NS]
    w_ref = refs[NS]
    g_refs = refs[NS + 1:2 * NS + 1]
    laux_ref = refs[2 * NS + 1]
    imp_ref = refs[2 * NS + 2]
    i = pl.program_id(0)
    bs = None
    for s in range(NS):
        g = jnp.dot(x_refs[s][...], w_ref[...], preferred_element_type=jnp.float32)
        g_refs[s][...] = g
        ps = jnp.sum(g, axis=0, keepdims=True)
        bs = ps if bs is None else bs + ps

    @pl.when(i == 0)
    def _():
        imp_ref[...] = bs

    @pl.when(i != 0)
    def _():
        imp_ref[...] = imp_ref[...] + bs

    @pl.when(i == nb - 1)
    def _():
        imp = imp_ref[...]
        mean = jnp.mean(imp)
        var = jnp.sum((imp - mean) ** 2) * (1.0 / (NE - 1))
        laux_ref[...] = jnp.reshape(V_IMP_F * var / (mean * mean), (1, 1))


def _tc_router(x, wt):
    n = x.shape[0]
    nh = n // NS
    nb = nh // BT

    def mk_in(s):
        return pl.BlockSpec((BT, IN_F), lambda i, s=s: (i + s * nb, 0))

    outs = pl.pallas_call(
        functools.partial(_tc_body, nb=nb),
        grid=(nb,),
        in_specs=[mk_in(s) for s in range(NS)]
        + [pl.BlockSpec((IN_F, NE), lambda i: (0, 0))],
        out_specs=[pl.BlockSpec((BT, NE), lambda i: (i, 0)) for _ in range(NS)]
        + [pl.BlockSpec((1, 1), lambda i: (0, 0))],
        out_shape=[jax.ShapeDtypeStruct((nh, NE), jnp.float32) for _ in range(NS)]
        + [jax.ShapeDtypeStruct((1, 1), jnp.float32)],
        scratch_shapes=[pltpu.VMEM((1, NE), jnp.float32)],
    )(*([x] * NS + [wt]))
    return list(outs[:NS]), outs[NS]


def _sc_gates(g_parts, n):
    per_w = n // NW
    flat_w = per_w * NE
    wps = NW // NS
    mesh = plsc.VectorSubcoreMesh(
        core_axis_name="c", subcore_axis_name="s",
        num_cores=SC_CORES, num_subcores=SC_SUBCORES,
    )

    @functools.partial(
        pl.kernel,
        mesh=mesh,
        out_type=jax.ShapeDtypeStruct((n * NE,), jnp.float32),
        scratch_types=[
            pltpu.VMEM((flat_w,), jnp.float32),
            pltpu.VMEM((flat_w,), jnp.float32),
        ],
        compiler_params=pltpu.CompilerParams(needs_layout_passes=False),
    )
    def k(*refs):
        g_refs = refs[:NS]
        out_hbm = refs[NS]
        g_v = refs[NS + 1]
        o_v = refs[NS + 2]
        wid = lax.axis_index("c") * SC_SUBCORES + lax.axis_index("s")
        s_id = wid // wps
        local_off = (wid % wps) * flat_w
        for s in range(NS):
            @pl.when(s_id == s)
            def _(s=s):
                pltpu.sync_copy(g_refs[s].at[pl.ds(local_off, flat_w)], g_v)

        def body(t, carry):
            tbase = t * (SC_LANES * NE) + lax.iota(jnp.int32, SC_LANES) * NE
            vs = [plsc.load_gather(g_v, [tbase + e]) for e in range(NE)]
            best = vs[0]
            bi = jnp.zeros((SC_LANES,), jnp.int32)
            for e in range(1, NE):
                gt = vs[e] > best
                best = jnp.where(gt, vs[e], best)
                bi = jnp.where(gt, jnp.full((SC_LANES,), e, jnp.int32), bi)
            m = jnp.maximum(best, BETA_F)
            e_fill = jnp.exp(BETA_F - m)
            e_top = jnp.exp(best - m)
            inv = 1.0 / ((NE - 1) * e_fill + e_top)
            g_fill = e_fill * inv
            g_top = e_top * inv
            for e in range(NE):
                oe = jnp.where(bi == e, g_top, g_fill)
                plsc.store_scatter(o_v, [tbase + e], oe)
            return carry

        lax.fori_loop(0, per_w // SC_LANES, body, 0)
        pltpu.sync_copy(o_v, out_hbm.at[pl.ds(wid * flat_w, flat_w)])

    return k(*[jnp.reshape(gp, (-1,)) for gp in g_parts])


def kernel(x, W_g):
    wt = W_g.T
    g_parts, laux = _tc_router(x, wt)
    n = x.shape[0]
    gates = jnp.reshape(_sc_gates(g_parts, n), (n, NE))
    return gates, jnp.reshape(laux, ())


# speedup vs baseline: 1.4640x; 1.4640x over previous
import functools

import jax
import jax.numpy as jnp
from jax import lax
from jax.experimental import pallas as pl
from jax.experimental.pallas import tpu as pltpu
from jax.experimental.pallas import tpu_sc as plsc

IN_F = 768
NE = 8
BETA_F = 1.0e6
V_IMP_F = 0.1

NS = 16
BT = 256

SC_CORES = 2
SC_SUBCORES = 16
SC_LANES = 16
NW = SC_CORES * SC_SUBCORES


def _tc_body(*refs, nb):
    x_refs = refs[:NS]
    w_ref = refs[NS]
    g_ref = refs[NS + 1]
    laux_ref = refs[NS + 2]
    imp_ref = refs[NS + 3]
    i = pl.program_id(0)
    bs = None
    gs = []
    for s in range(NS):
        g = jnp.dot(x_refs[s][...], w_ref[...], preferred_element_type=jnp.float32)
        gs.append(g)
        ps = jnp.sum(g, axis=0, keepdims=True)
        bs = ps if bs is None else bs + ps
    g_ref[...] = jnp.concatenate(gs, axis=1)

    @pl.when(i == 0)
    def _():
        imp_ref[...] = bs

    @pl.when(i != 0)
    def _():
        imp_ref[...] = imp_ref[...] + bs

    @pl.when(i == nb - 1)
    def _():
        imp = imp_ref[...]
        mean = jnp.mean(imp)
        var = jnp.sum((imp - mean) ** 2) * (1.0 / (NE - 1))
        laux_ref[...] = jnp.reshape(V_IMP_F * var / (mean * mean), (1, 1))


def _tc_router(x, wt):
    n = x.shape[0]
    nh = n // NS
    nb = nh // BT

    def mk_in(s):
        return pl.BlockSpec((BT, IN_F), lambda i, s=s: (i + s * nb, 0))

    g, laux = pl.pallas_call(
        functools.partial(_tc_body, nb=nb),
        grid=(nb,),
        in_specs=[mk_in(s) for s in range(NS)]
        + [pl.BlockSpec((IN_F, NE), lambda i: (0, 0))],
        out_specs=[
            pl.BlockSpec((BT, NS * NE), lambda i: (i, 0)),
            pl.BlockSpec((1, 1), lambda i: (0, 0)),
        ],
        out_shape=[
            jax.ShapeDtypeStruct((nh, NS * NE), jnp.float32),
            jax.ShapeDtypeStruct((1, 1), jnp.float32),
        ],
        scratch_shapes=[pltpu.VMEM((1, NE), jnp.float32)],
    )(*([x] * NS + [wt]))
    return g, laux


def _sc_gates(gflat, n):
    per_w = n // NW
    flat_w = per_w * NE
    rows_w = per_w // NS
    nh = n // NS
    mesh = plsc.VectorSubcoreMesh(
        core_axis_name="c", subcore_axis_name="s",
        num_cores=SC_CORES, num_subcores=SC_SUBCORES,
    )

    @functools.partial(
        pl.kernel,
        mesh=mesh,
        out_type=jax.ShapeDtypeStruct((n * NE,), jnp.float32),
        scratch_types=[
            pltpu.VMEM((flat_w,), jnp.float32),
            pltpu.VMEM((flat_w,), jnp.float32),
            pltpu.SemaphoreType.DMA,
        ],
        compiler_params=pltpu.CompilerParams(needs_layout_passes=False),
    )
    def k(g_hbm, out_hbm, g_v, o_v, sem):
        wid = lax.axis_index("c") * SC_SUBCORES + lax.axis_index("s")
        pltpu.sync_copy(g_hbm.at[pl.ds(wid * flat_w, flat_w)], g_v)

        def body(t, carry):
            s = t // (rows_w // SC_LANES)
            r0 = (t % (rows_w // SC_LANES)) * SC_LANES
            iot = lax.iota(jnp.int32, SC_LANES)
            tbase = (r0 + iot) * (NS * NE) + s * NE
            vs = [plsc.load_gather(g_v, [tbase + e]) for e in range(NE)]
            best = vs[0]
            bi = jnp.zeros((SC_LANES,), jnp.int32)
            for e in range(1, NE):
                gt = vs[e] > best
                best = jnp.where(gt, vs[e], best)
                bi = jnp.where(gt, jnp.full((SC_LANES,), e, jnp.int32), bi)
            m = jnp.maximum(best, BETA_F)
            e_fill = jnp.exp(BETA_F - m)
            e_top = jnp.exp(best - m)
            inv = 1.0 / ((NE - 1) * e_fill + e_top)
            g_fill = e_fill * inv
            g_top = e_top * inv
            obase = s * (rows_w * NE) + (r0 + iot) * NE
            for e in range(NE):
                oe = jnp.where(bi == e, g_top, g_fill)
                plsc.store_scatter(o_v, [obase + e], oe)
            return carry

        lax.fori_loop(0, per_w // SC_LANES, body, 0)
        slab = rows_w * NE
        handles = [
            pltpu.async_copy(
                o_v.at[pl.ds(s * slab, slab)],
                out_hbm.at[pl.ds((s * nh + wid * rows_w) * NE, slab)],
                sem,
            )
            for s in range(NS)
        ]
        for h in handles:
            h.wait()

    return k(gflat)


def kernel(x, W_g):
    wt = W_g.T
    g, laux = _tc_router(x, wt)
    n = x.shape[0]
    gates = jnp.reshape(_sc_gates(jnp.reshape(g, (-1,)), n), (n, NE))
    return gates, jnp.reshape(laux, ())
